# baseline (speedup 1.0000x reference)
import jax
import jax.numpy as jnp
from jax.experimental import pallas as pl
from jax.experimental.pallas import tpu as pltpu


def _body(ti_ref, wi_ref, tt_ref, wt_ref, wkt_ref, wwk_ref, nt_ref, wn_ref,
          out_ref):
    b = pl.program_id(0)
    T = out_ref.shape[1]
    s = jnp.dot(nt_ref[:], wn_ref[:], preferred_element_type=jnp.float32)
    for t in range(T):
        ti = ti_ref[b, t]
        wi = wi_ref[b, t]
        trow = tt_ref[pl.ds(ti, 1), :]
        wrow = wkt_ref[pl.ds(wi, 1), :]
        v = (jnp.dot(trow, wt_ref[:], preferred_element_type=jnp.float32)
             + jnp.dot(wrow, wwk_ref[:], preferred_element_type=jnp.float32))
        out_ref[0, t] = s + v


def kernel(time, weekday, time_table, W_time, weekday_table, W_weekday,
           node_table, W_node):
    B, T, _ = time.shape
    N, _ = node_table.shape
    D = W_node.shape[1]
    ti = time.reshape(B, T).astype(jnp.int32)
    wi = weekday.reshape(B, T).astype(jnp.int32)

    smem = pl.BlockSpec(memory_space=pltpu.SMEM)

    def full(shape):
        return pl.BlockSpec(shape, lambda b: (0,) * len(shape))

    out = pl.pallas_call(
        _body,
        grid=(B,),
        in_specs=[
            smem,
            smem,
            full(time_table.shape),
            full(W_time.shape),
            full(weekday_table.shape),
            full(W_weekday.shape),
            full(node_table.shape),
            full(W_node.shape),
        ],
        out_specs=pl.BlockSpec((1, T, N, D), lambda b: (b, 0, 0, 0)),
        out_shape=jax.ShapeDtypeStruct((B, T, N, D), jnp.float32),
        compiler_params=pltpu.CompilerParams(
            dimension_semantics=("arbitrary",)),
    )(ti, wi, time_table, W_time, weekday_table, W_weekday, node_table,
      W_node)
    return out


# speedup vs baseline: 1.0084x; 1.0084x over previous
import jax
import jax.numpy as jnp
from jax import lax
from jax.experimental import pallas as pl
from jax.experimental.pallas import tpu as pltpu


def _prep_body(ti_ref, wi_ref, tt_ref, wt_ref, wkt_ref, wwk_ref, nt_ref,
               wn_ref, s_ref, tv_ref):
    f32 = jnp.float32
    s_ref[:] = jnp.dot(nt_ref[:], wn_ref[:], preferred_element_type=f32)
    tproj = jnp.dot(tt_ref[:], wt_ref[:], preferred_element_type=f32)
    wproj = jnp.dot(wkt_ref[:], wwk_ref[:], preferred_element_type=f32)
    BT = ti_ref.shape[0]
    V_t = tt_ref.shape[0]
    V_w = wkt_ref.shape[0]
    oh_t = (ti_ref[:] == lax.broadcasted_iota(jnp.int32, (BT, V_t), 1)
            ).astype(f32)
    oh_w = (wi_ref[:] == lax.broadcasted_iota(jnp.int32, (BT, V_w), 1)
            ).astype(f32)
    tv_ref[:] = (jnp.dot(oh_t, tproj, preferred_element_type=f32)
                 + jnp.dot(oh_w, wproj, preferred_element_type=f32))


def _stream_body(s_ref, tv_ref, out_ref):
    b = pl.program_id(0)
    T = out_ref.shape[1]
    s = s_ref[:]
    for t in range(T):
        out_ref[0, t] = s + tv_ref[b * T + t, :]


def kernel(time, weekday, time_table, W_time, weekday_table, W_weekday,
           node_table, W_node):
    B, T, _ = time.shape
    N, _ = node_table.shape
    D = W_node.shape[1]
    ti = time.reshape(B * T, 1).astype(jnp.int32)
    wi = weekday.reshape(B * T, 1).astype(jnp.int32)
    wkt = jnp.pad(weekday_table, ((0, 1), (0, 0)))

    def full(shape):
        return pl.BlockSpec(shape, lambda: (0,) * len(shape))

    s, tv = pl.pallas_call(
        _prep_body,
        in_specs=[full(ti.shape), full(wi.shape), full(time_table.shape),
                  full(W_time.shape), full(wkt.shape), full(W_weekday.shape),
                  full(node_table.shape), full(W_node.shape)],
        out_specs=[full((N, D)), full((B * T, D))],
        out_shape=[jax.ShapeDtypeStruct((N, D), jnp.float32),
                   jax.ShapeDtypeStruct((B * T, D), jnp.float32)],
    )(ti, wi, time_table, W_time, wkt, W_weekday, node_table, W_node)

    out = pl.pallas_call(
        _stream_body,
        grid=(B,),
        in_specs=[
            pl.BlockSpec((N, D), lambda b: (0, 0)),
            pl.BlockSpec((B * T, D), lambda b: (0, 0)),
        ],
        out_specs=pl.BlockSpec((1, T, N, D), lambda b: (b, 0, 0, 0)),
        out_shape=jax.ShapeDtypeStruct((B, T, N, D), jnp.float32),
        compiler_params=pltpu.CompilerParams(
            dimension_semantics=("arbitrary",)),
    )(s, tv)
    return out


# speedup vs baseline: 1.1333x; 1.1239x over previous
import jax
import jax.numpy as jnp
from jax import lax
from jax.experimental import pallas as pl
from jax.experimental.pallas import tpu as pltpu


def _prep_body(ti_ref, wi_ref, tt_ref, wt_ref, wkt_ref, wwk_ref, nt_ref,
               wn_ref, s_ref, tv_ref):
    f32 = jnp.float32
    s_ref[:] = jnp.dot(nt_ref[:], wn_ref[:], preferred_element_type=f32)
    tproj = jnp.dot(tt_ref[:], wt_ref[:], preferred_element_type=f32)
    wproj = jnp.dot(wkt_ref[:], wwk_ref[:], preferred_element_type=f32)
    BT = ti_ref.shape[0]
    V_t = tt_ref.shape[0]
    V_w = wkt_ref.shape[0]
    oh_t = (ti_ref[:] == lax.broadcasted_iota(jnp.int32, (BT, V_t), 1)
            ).astype(f32)
    oh_w = (wi_ref[:] == lax.broadcasted_iota(jnp.int32, (BT, V_w), 1)
            ).astype(f32)
    tv_ref[:] = (jnp.dot(oh_t, tproj, preferred_element_type=f32)
                 + jnp.dot(oh_w, wproj, preferred_element_type=f32))


def _stream_body(s_ref, tv_ref, out_ref):
    p = pl.program_id(0)
    BB = out_ref.shape[0]
    T = out_ref.shape[1]
    s = s_ref[:]
    for bb in range(BB):
        for t in range(T):
            out_ref[bb, t] = s + tv_ref[(p * BB + bb) * T + t, :]


def kernel(time, weekday, time_table, W_time, weekday_table, W_weekday,
           node_table, W_node):
    B, T, _ = time.shape
    N, _ = node_table.shape
    D = W_node.shape[1]
    ti = time.reshape(B * T, 1).astype(jnp.int32)
    wi = weekday.reshape(B * T, 1).astype(jnp.int32)
    wkt = jnp.pad(weekday_table, ((0, 1), (0, 0)))

    def full(shape):
        return pl.BlockSpec(shape, lambda: (0,) * len(shape))

    s, tv = pl.pallas_call(
        _prep_body,
        in_specs=[full(ti.shape), full(wi.shape), full(time_table.shape),
                  full(W_time.shape), full(wkt.shape), full(W_weekday.shape),
                  full(node_table.shape), full(W_node.shape)],
        out_specs=[full((N, D)), full((B * T, D))],
        out_shape=[jax.ShapeDtypeStruct((N, D), jnp.float32),
                   jax.ShapeDtypeStruct((B * T, D), jnp.float32)],
    )(ti, wi, time_table, W_time, wkt, W_weekday, node_table, W_node)

    BB = 8
    out = pl.pallas_call(
        _stream_body,
        grid=(B // BB,),
        in_specs=[
            pl.BlockSpec((N, D), lambda b: (0, 0)),
            pl.BlockSpec((B * T, D), lambda b: (0, 0)),
        ],
        out_specs=pl.BlockSpec((BB, T, N, D), lambda b: (b, 0, 0, 0)),
        out_shape=jax.ShapeDtypeStruct((B, T, N, D), jnp.float32),
        compiler_params=pltpu.CompilerParams(
            dimension_semantics=("arbitrary",)),
    )(s, tv)
    return out


# speedup vs baseline: 1.1367x; 1.0030x over previous
import jax
import jax.numpy as jnp
from jax import lax
from jax.experimental import pallas as pl
from jax.experimental.pallas import tpu as pltpu


def _prep_body(ti_ref, wi_ref, tt_ref, wt_ref, wkt_ref, wwk_ref, nt_ref,
               wn_ref, s_ref, tv_ref):
    f32 = jnp.float32
    s_ref[:] = jnp.dot(nt_ref[:], wn_ref[:], preferred_element_type=f32)
    tproj = jnp.dot(tt_ref[:], wt_ref[:], preferred_element_type=f32)
    wproj = jnp.dot(wkt_ref[:], wwk_ref[:], preferred_element_type=f32)
    BT = ti_ref.shape[0]
    V_t = tt_ref.shape[0]
    V_w = wkt_ref.shape[0]
    oh_t = (ti_ref[:] == lax.broadcasted_iota(jnp.int32, (BT, V_t), 1)
            ).astype(f32)
    oh_w = (wi_ref[:] == lax.broadcasted_iota(jnp.int32, (BT, V_w), 1)
            ).astype(f32)
    tv_ref[:] = (jnp.dot(oh_t, tproj, preferred_element_type=f32)
                 + jnp.dot(oh_w, wproj, preferred_element_type=f32))


_NBUF = 4
_CH = 4


def _stream_body(s_ref, tv_ref, out_ref, buf, sems):
    p = pl.program_id(0)
    n = pl.num_programs(0)
    T = buf.shape[2]
    s = s_ref[:]
    slot = jax.lax.rem(p, _NBUF)

    @pl.when(p >= _NBUF)
    def _():
        pltpu.make_async_copy(buf.at[slot], out_ref.at[pl.ds(0, _CH)],
                              sems.at[slot]).wait()

    for bb in range(_CH):
        for t in range(T):
            buf[slot, bb, t] = s + tv_ref[(p * _CH + bb) * T + t, :]

    pltpu.make_async_copy(buf.at[slot], out_ref.at[pl.ds(p * _CH, _CH)],
                          sems.at[slot]).start()

    @pl.when(p == n - 1)
    def _():
        for k in range(_NBUF):
            pltpu.make_async_copy(buf.at[k], out_ref.at[pl.ds(0, _CH)],
                                  sems.at[k]).wait()


def kernel(time, weekday, time_table, W_time, weekday_table, W_weekday,
           node_table, W_node):
    B, T, _ = time.shape
    N, _ = node_table.shape
    D = W_node.shape[1]
    ti = time.reshape(B * T, 1).astype(jnp.int32)
    wi = weekday.reshape(B * T, 1).astype(jnp.int32)
    wkt = jnp.pad(weekday_table, ((0, 1), (0, 0)))

    def full(shape):
        return pl.BlockSpec(shape, lambda: (0,) * len(shape))

    s, tv = pl.pallas_call(
        _prep_body,
        in_specs=[full(ti.shape), full(wi.shape), full(time_table.shape),
                  full(W_time.shape), full(wkt.shape), full(W_weekday.shape),
                  full(node_table.shape), full(W_node.shape)],
        out_specs=[full((N, D)), full((B * T, D))],
        out_shape=[jax.ShapeDtypeStruct((N, D), jnp.float32),
                   jax.ShapeDtypeStruct((B * T, D), jnp.float32)],
    )(ti, wi, time_table, W_time, wkt, W_weekday, node_table, W_node)

    out = pl.pallas_call(
        _stream_body,
        grid=(B // _CH,),
        in_specs=[
            pl.BlockSpec((N, D), lambda b: (0, 0)),
            pl.BlockSpec((B * T, D), lambda b: (0, 0)),
        ],
        out_specs=pl.BlockSpec(memory_space=pl.ANY),
        out_shape=jax.ShapeDtypeStruct((B, T, N, D), jnp.float32),
        scratch_shapes=[
            pltpu.VMEM((_NBUF, _CH, T, N, D), jnp.float32),
            pltpu.SemaphoreType.DMA((_NBUF,)),
        ],
        compiler_params=pltpu.CompilerParams(
            dimension_semantics=("arbitrary",)),
    )(s, tv)
    return out


# speedup vs baseline: 3.3901x; 2.9824x over previous
import jax
import jax.numpy as jnp
from jax import lax
from jax.experimental import pallas as pl
from jax.experimental.pallas import tpu as pltpu


def _dot0(a, b):
    return lax.dot_general(a, b, (((0,), (0,)), ((), ())),
                           preferred_element_type=jnp.float32)


def _body(ti_ref, wi_ref, ttT_ref, wt_ref, wkt_ref, wwk_ref, ntT_ref, wn_ref,
          out_ref, s_scr):
    t = pl.program_id(0)
    N = out_ref.shape[1]
    B = out_ref.shape[2]
    Vt = ttT_ref.shape[1]

    tproj = _dot0(ttT_ref[:], wt_ref[:])
    wproj = jnp.dot(wkt_ref[:], wwk_ref[:],
                    preferred_element_type=jnp.float32)
    wproj8 = jnp.concatenate(
        [wproj, jnp.zeros((1, wproj.shape[1]), jnp.float32)], axis=0)
    s_scr[:] = _dot0(ntT_ref[:], wn_ref[:])

    ti = ti_ref[t]
    wi = wi_ref[t]
    oh_t = (ti[None, :] == lax.broadcasted_iota(jnp.int32, (Vt, B), 0)
            ).astype(jnp.float32)
    oh_w = (wi[None, :] == lax.broadcasted_iota(jnp.int32, (8, B), 0)
            ).astype(jnp.float32)
    tv = _dot0(oh_t, tproj) + _dot0(oh_w, wproj8)

    for n in range(N):
        out_ref[0, n] = tv + s_scr[n, :]


def kernel(time, weekday, time_table, W_time, weekday_table, W_weekday,
           node_table, W_node):
    B, T, _ = time.shape
    N, _ = node_table.shape
    D = W_node.shape[1]
    ti = time.reshape(B, T).T.astype(jnp.int32)
    wi = weekday.reshape(B, T).T.astype(jnp.int32)

    def full(shape):
        return pl.BlockSpec(shape, lambda t: (0,) * len(shape))

    out = pl.pallas_call(
        _body,
        grid=(T,),
        in_specs=[full(ti.shape), full(wi.shape),
                  full((time_table.shape[1], time_table.shape[0])),
                  full(W_time.shape), full(weekday_table.shape),
                  full(W_weekday.shape),
                  full((node_table.shape[1], node_table.shape[0])),
                  full(W_node.shape)],
        out_specs=pl.BlockSpec((1, N, B, D), lambda t: (t, 0, 0, 0)),
        out_shape=jax.ShapeDtypeStruct((T, N, B, D), jnp.float32),
        scratch_shapes=[pltpu.VMEM((N, D), jnp.float32)],
        compiler_params=pltpu.CompilerParams(
            dimension_semantics=("arbitrary",)),
    )(ti, wi, time_table.T, W_time, weekday_table, W_weekday, node_table.T,
      W_node)
    return jnp.transpose(out, (2, 0, 1, 3))


# speedup vs baseline: 3.3935x; 1.0010x over previous
import jax
import jax.numpy as jnp
from jax import lax
from jax.experimental import pallas as pl
from jax.experimental.pallas import tpu as pltpu


def _dot0(a, b):
    return lax.dot_general(a, b, (((0,), (0,)), ((), ())),
                           preferred_element_type=jnp.float32)


def _body(idx_ref, ttT_ref, wt_ref, wkt_ref, wwk_ref, ntT_ref, wn_ref,
          out_ref, s_scr):
    t = pl.program_id(0)
    T = pl.num_programs(0)
    N = out_ref.shape[1]
    B = out_ref.shape[2]
    Vt = ttT_ref.shape[1]

    tproj = _dot0(ttT_ref[:], wt_ref[:])
    wproj = jnp.dot(wkt_ref[:], wwk_ref[:],
                    preferred_element_type=jnp.float32)
    wproj8 = jnp.concatenate(
        [wproj, jnp.zeros((1, wproj.shape[1]), jnp.float32)], axis=0)
    s_scr[:] = _dot0(ntT_ref[:], wn_ref[:])

    ti = idx_ref[t]
    wi = idx_ref[T + t]
    oh_t = (ti[None, :] == lax.broadcasted_iota(jnp.int32, (Vt, B), 0)
            ).astype(jnp.float32)
    oh_w = (wi[None, :] == lax.broadcasted_iota(jnp.int32, (8, B), 0)
            ).astype(jnp.float32)
    tv = _dot0(oh_t, tproj) + _dot0(oh_w, wproj8)

    for n in range(N):
        out_ref[0, n] = tv + s_scr[n, :]


def kernel(time, weekday, time_table, W_time, weekday_table, W_weekday,
           node_table, W_node):
    B, T, _ = time.shape
    N, _ = node_table.shape
    D = W_node.shape[1]
    idx = jnp.concatenate(
        [time.reshape(B, T).T, weekday.reshape(B, T).T],
        axis=0).astype(jnp.int32)

    def full(shape):
        return pl.BlockSpec(shape, lambda t: (0,) * len(shape))

    out = pl.pallas_call(
        _body,
        grid=(T,),
        in_specs=[full(idx.shape),
                  full((time_table.shape[1], time_table.shape[0])),
                  full(W_time.shape), full(weekday_table.shape),
                  full(W_weekday.shape),
                  full((node_table.shape[1], node_table.shape[0])),
                  full(W_node.shape)],
        out_specs=pl.BlockSpec((1, N, B, D), lambda t: (t, 0, 0, 0)),
        out_shape=jax.ShapeDtypeStruct((T, N, B, D), jnp.float32),
        scratch_shapes=[pltpu.VMEM((N, D), jnp.float32)],
        compiler_params=pltpu.CompilerParams(
            dimension_semantics=("arbitrary",)),
    )(idx, time_table.T, W_time, weekday_table, W_weekday, node_table.T,
      W_node)
    return jnp.transpose(out, (2, 0, 1, 3))
